# baseline (speedup 1.0000x reference)
import functools

import jax
import jax.numpy as jnp
from jax import lax
from jax.experimental import pallas as pl
from jax.experimental.pallas import tpu as pltpu
from jax.experimental.pallas import tpu_sc as plsc

_B, _N, _P, _C, _S = 4, 8192, 2048, 64, 32
_R2 = 0.2 * 0.2
_R2P = (0.5 * 0.2) ** 2
_WPB = 8
_QPW = _P // _WPB
_WPC = _C // _WPB
_PS = _P * _S
_NCH = 3 + _C
_CH = 4096
_SPAN = _QPW * _S

_F_PTX = 0
_F_PTY = _N
_F_PTZ = 2 * _N
_F_STD = 3 * _N
_F_Q = 3 * _N + 3 * _SPAN
_F_STG = 0
_F_SIZE = max(3 * _N + 3 * _SPAN + 3 * _QPW, 2 * 4 * _CH)

_I_PTC = 0
_I_QC = _N
_I_WIDE = _N + _QPW
_I_IDXC = _N + _QPW + 64 * _QPW
_I_SIZE = _N + _QPW + 64 * _QPW + _SPAN
_I_IB = (0, _CH)

_mesh = plsc.VectorSubcoreMesh(core_axis_name="c", subcore_axis_name="s")


@functools.partial(
    pl.kernel,
    out_type=jax.ShapeDtypeStruct((_B * _NCH * _PS,), jnp.float32),
    mesh=_mesh,
    compiler_params=pltpu.CompilerParams(needs_layout_passes=False),
    scratch_types=[
        pltpu.VMEM((_F_SIZE,), jnp.float32),
        pltpu.VMEM((4, _N), jnp.float32),
        pltpu.VMEM((_I_SIZE,), jnp.int32),
        pltpu.VMEM_SHARED((2 * _PS,), jnp.int32),
        pltpu.SemaphoreType.DMA,
        pltpu.SemaphoreType.DMA,
        pltpu.SemaphoreType.DMA,
        pltpu.SemaphoreType.DMA,
    ],
)
def _gag(xs_hbm, ys_hbm, zs_hbm, comp_hbm, nxs_hbm, nys_hbm, nzs_hbm,
         ncomp_hbm, feat_hbm, out_hbm, fa, tabs, ia, sh_idx,
         osem0, osem1, isem0, isem1):
    cid = lax.axis_index("c")
    sid = lax.axis_index("s")
    lb = sid // _WPB
    b = cid * 2 + lb
    grp = sid % _WPB
    q0 = b * _P + grp * _QPW

    pltpu.sync_copy(xs_hbm.at[pl.ds(b * _N, _N)], fa.at[pl.ds(_F_PTX, _N)])
    pltpu.sync_copy(ys_hbm.at[pl.ds(b * _N, _N)], fa.at[pl.ds(_F_PTY, _N)])
    pltpu.sync_copy(zs_hbm.at[pl.ds(b * _N, _N)], fa.at[pl.ds(_F_PTZ, _N)])
    pltpu.sync_copy(comp_hbm.at[pl.ds(b * _N, _N)], ia.at[pl.ds(_I_PTC, _N)])
    pltpu.sync_copy(nxs_hbm.at[pl.ds(q0, _QPW)],
                    fa.at[pl.ds(_F_Q, _QPW)])
    pltpu.sync_copy(nys_hbm.at[pl.ds(q0, _QPW)],
                    fa.at[pl.ds(_F_Q + _QPW, _QPW)])
    pltpu.sync_copy(nzs_hbm.at[pl.ds(q0, _QPW)],
                    fa.at[pl.ds(_F_Q + 2 * _QPW, _QPW)])
    pltpu.sync_copy(ncomp_hbm.at[pl.ds(q0, _QPW)], ia.at[pl.ds(_I_QC, _QPW)])

    lane = lax.iota(jnp.int32, 16)
    zeros16 = jnp.zeros((16,), jnp.int32)
    ones16 = jnp.ones((16,), jnp.int32)
    unroll = 4
    qpack = 8
    stride = _QPW // qpack

    def per_query(p, carry):
        qs = tuple(p + i * stride for i in range(qpack))
        qxyzc = []
        ccs0 = []
        for q in qs:
            spl = jnp.full((16,), q, jnp.int32)
            sbase = _I_WIDE + q * 64
            qxyzc.append((
                plsc.load_gather(fa.at[pl.ds(_F_Q, _QPW)], [spl]),
                plsc.load_gather(fa.at[pl.ds(_F_Q + _QPW, _QPW)], [spl]),
                plsc.load_gather(fa.at[pl.ds(_F_Q + 2 * _QPW, _QPW)], [spl]),
                plsc.load_gather(ia.at[pl.ds(_I_QC, _QPW)], [spl]),
                jnp.full((16,), sbase + 31, jnp.int32)))
            ccs0.append(jnp.full((16,), sbase - 1, jnp.int32))
            ia[pl.ds(sbase, 16)] = zeros16

        def chunk(j, ccs):
            out = []
            for u in range(unroll):
                off = j * (16 * unroll) + u * 16
                px = fa[pl.ds(_F_PTX + off, 16)]
                py = fa[pl.ds(_F_PTY + off, 16)]
                pz = fa[pl.ds(_F_PTZ + off, 16)]
                pc = ia[pl.ds(_I_PTC + off, 16)]
                gidx = lane + off
                out = []
                for (qx, qy, qz, qc, cap_v), ccb in zip(qxyzc, ccs):
                    dx = px - qx
                    dy = py - qy
                    dz = pz - qz
                    d2 = dx * dx + dy * dy + dz * dz
                    thr = jnp.where(pc == qc, _R2, _R2P).astype(jnp.float32)
                    elig = d2 < thr
                    incl = plsc.cumsum(ones16, mask=elig)
                    plsc.store_scatter(ia, [ccb + incl], gidx, mask=elig)
                    ccb = jnp.minimum(
                        ccb + plsc.all_reduce_population_count(elig), cap_v)
                    out.append(ccb)
                ccs = tuple(out)
            return ccs

        ccs = lax.fori_loop(0, _N // (16 * unroll), chunk, tuple(ccs0))
        for q, (qx, qy, qz, qc, _cap), ccb in zip(qs, qxyzc, ccs):
            base = q * _S
            sbase = _I_WIDE + q * 64
            cntc = ccb - jnp.full((16,), sbase - 1, jnp.int32)
            pad = plsc.load_gather(ia, [jnp.full((16,), sbase, jnp.int32)])
            for h in range(2):
                lids = lane + h * 16
                iv = jnp.where(lids >= cntc,
                               pad, ia[pl.ds(sbase + h * 16, 16)])
                ia[pl.ds(_I_IDXC + base + h * 16, 16)] = iv
                fa[pl.ds(_F_STD + base + h * 16, 16)] = (
                    plsc.load_gather(fa.at[pl.ds(_F_PTX, _N)], [iv]) - qx)
                fa[pl.ds(_F_STD + _SPAN + base + h * 16, 16)] = (
                    plsc.load_gather(fa.at[pl.ds(_F_PTY, _N)], [iv]) - qy)
                fa[pl.ds(_F_STD + 2 * _SPAN + base + h * 16, 16)] = (
                    plsc.load_gather(fa.at[pl.ds(_F_PTZ, _N)], [iv]) - qz)
        return carry

    lax.fori_loop(0, stride, per_query, 0)

    w0 = lb * _PS + grp * _SPAN
    pltpu.sync_copy(ia.at[pl.ds(_I_IDXC, _SPAN)], sh_idx.at[pl.ds(w0, _SPAN)])
    for d in range(3):
        pltpu.sync_copy(
            fa.at[pl.ds(_F_STD + d * _SPAN, _SPAN)],
            out_hbm.at[pl.ds((b * _NCH + d) * _PS + grp * _SPAN, _SPAN)])
    plsc.subcore_barrier()

    c0 = grp * _WPC
    nchunks = _PS // _CH
    osems = (osem0, osem1)
    isems = (isem0, isem1)
    chsel = [jnp.full((16,), ch, jnp.int32) for ch in range(4)]
    pending_out = [None, None]
    pending_idx = [None, None]

    def idx_copy(k):
        return pltpu.make_async_copy(
            sh_idx.at[pl.ds(lb * _PS + k * _CH, _CH)],
            ia.at[pl.ds(_I_IB[k % 2], _CH)], isems[k % 2])

    for chalf in range(2):
        ch0 = c0 + chalf * 4
        pltpu.sync_copy(feat_hbm.at[b, pl.ds(ch0, 4), :], tabs)
        if chalf == 0:
            pending_idx[0] = idx_copy(0)
            pending_idx[0].start()
        for k in range(nchunks):
            buf = k % 2
            k0 = k * _CH
            ib = _I_IB[buf]
            stg = _F_STG + buf * 4 * _CH
            nk = k + 1 if k + 1 < nchunks else (0 if chalf == 0 else None)
            if nk is not None:
                pending_idx[nk % 2] = idx_copy(nk)
                pending_idx[nk % 2].start()
            pending_idx[buf].wait()
            if pending_out[buf] is not None:
                for c in pending_out[buf]:
                    c.wait()

            def inner(t, c2, ib=ib, stg=stg):
                for u in range(4):
                    o = t * 64 + u * 16
                    iv = ia[pl.ds(ib + o, 16)]
                    for ch in range(4):
                        fa[pl.ds(stg + ch * _CH + o, 16)] = (
                            plsc.load_gather(tabs, [chsel[ch], iv]))
                return c2

            lax.fori_loop(0, _CH // 64, inner, 0)
            outs = []
            for ch in range(4):
                c = pltpu.make_async_copy(
                    fa.at[pl.ds(stg + ch * _CH, _CH)],
                    out_hbm.at[pl.ds(
                        (b * _NCH + 3 + ch0 + ch) * _PS + k0, _CH)],
                    osems[buf])
                c.start()
                outs.append(c)
            pending_out[buf] = outs
        for buf2 in range(2):
            if pending_out[buf2] is not None:
                for c in pending_out[buf2]:
                    c.wait()
                pending_out[buf2] = None


def kernel(xyz, new_xyz, components, new_components, features):
    xyz32 = xyz.astype(jnp.float32)
    nxyz32 = new_xyz.astype(jnp.float32)
    xs = xyz32[:, :, 0].reshape(-1)
    ys = xyz32[:, :, 1].reshape(-1)
    zs = xyz32[:, :, 2].reshape(-1)
    nxs = nxyz32[:, :, 0].reshape(-1)
    nys = nxyz32[:, :, 1].reshape(-1)
    nzs = nxyz32[:, :, 2].reshape(-1)
    comp = components.reshape(-1).astype(jnp.int32)
    ncomp = new_components.reshape(-1).astype(jnp.int32)
    out = _gag(xs, ys, zs, comp, nxs, nys, nzs, ncomp,
               features.astype(jnp.float32))
    return out.reshape(_B, _NCH, _P, _S)


# speedup vs baseline: 1.0208x; 1.0208x over previous
import functools

import jax
import jax.numpy as jnp
from jax import lax
from jax.experimental import pallas as pl
from jax.experimental.pallas import tpu as pltpu
from jax.experimental.pallas import tpu_sc as plsc

_B, _N, _P, _C, _S = 4, 8192, 2048, 64, 32
_R2 = 0.2 * 0.2
_R2P = (0.5 * 0.2) ** 2
_WPB = 8
_QPW = _P // _WPB
_WPC = _C // _WPB
_PS = _P * _S
_NCH = 3 + _C
_CH = 4096
_SPAN = _QPW * _S

_F_PTX = 0
_F_PTY = _N
_F_PTZ = 2 * _N
_F_STD = 3 * _N
_F_Q = 3 * _N + 3 * _SPAN
_F_TABS = 0
_F_STG = 4 * _N
_F_SIZE = max(3 * _N + 3 * _SPAN + 3 * _QPW, 4 * _N + 2 * 4 * _CH)

_I_PTC = 0
_I_QC = _N
_I_WIDE = _N + _QPW
_I_IDXC = _N + _QPW + 64 * _QPW
_I_SIZE = _N + _QPW + 64 * _QPW + _SPAN
_I_IB = (0, _CH)

_mesh = plsc.VectorSubcoreMesh(core_axis_name="c", subcore_axis_name="s")


@functools.partial(
    pl.kernel,
    out_type=jax.ShapeDtypeStruct((_B * _NCH * _PS,), jnp.float32),
    mesh=_mesh,
    compiler_params=pltpu.CompilerParams(needs_layout_passes=False),
    scratch_types=[
        pltpu.VMEM((_F_SIZE,), jnp.float32),
        pltpu.VMEM((_I_SIZE,), jnp.int32),
        pltpu.VMEM_SHARED((2 * _PS,), jnp.int32),
        pltpu.SemaphoreType.DMA,
        pltpu.SemaphoreType.DMA,
        pltpu.SemaphoreType.DMA,
        pltpu.SemaphoreType.DMA,
    ],
)
def _gag(xs_hbm, ys_hbm, zs_hbm, comp_hbm, nxs_hbm, nys_hbm, nzs_hbm,
         ncomp_hbm, feat_hbm, out_hbm, fa, ia, sh_idx,
         osem0, osem1, isem0, isem1):
    cid = lax.axis_index("c")
    sid = lax.axis_index("s")
    lb = sid // _WPB
    b = cid * 2 + lb
    grp = sid % _WPB
    q0 = b * _P + grp * _QPW

    in_copies = [
        pltpu.make_async_copy(xs_hbm.at[pl.ds(b * _N, _N)],
                              fa.at[pl.ds(_F_PTX, _N)], isem0),
        pltpu.make_async_copy(ys_hbm.at[pl.ds(b * _N, _N)],
                              fa.at[pl.ds(_F_PTY, _N)], isem0),
        pltpu.make_async_copy(zs_hbm.at[pl.ds(b * _N, _N)],
                              fa.at[pl.ds(_F_PTZ, _N)], isem0),
        pltpu.make_async_copy(comp_hbm.at[pl.ds(b * _N, _N)],
                              ia.at[pl.ds(_I_PTC, _N)], isem0),
        pltpu.make_async_copy(nxs_hbm.at[pl.ds(q0, _QPW)],
                              fa.at[pl.ds(_F_Q, _QPW)], isem0),
        pltpu.make_async_copy(nys_hbm.at[pl.ds(q0, _QPW)],
                              fa.at[pl.ds(_F_Q + _QPW, _QPW)], isem0),
        pltpu.make_async_copy(nzs_hbm.at[pl.ds(q0, _QPW)],
                              fa.at[pl.ds(_F_Q + 2 * _QPW, _QPW)], isem0),
        pltpu.make_async_copy(ncomp_hbm.at[pl.ds(q0, _QPW)],
                              ia.at[pl.ds(_I_QC, _QPW)], isem0),
    ]
    for c in in_copies:
        c.start()
    for c in in_copies:
        c.wait()

    lane = lax.iota(jnp.int32, 16)
    zeros16 = jnp.zeros((16,), jnp.int32)
    ones16 = jnp.ones((16,), jnp.int32)
    unroll = 4
    qpack = 8
    stride = _QPW // qpack

    def per_query(p, carry):
        qs = tuple(p + i * stride for i in range(qpack))
        qxyzc = []
        ccs0 = []
        for q in qs:
            spl = jnp.full((16,), q, jnp.int32)
            sbase = _I_WIDE + q * 64
            qxyzc.append((
                plsc.load_gather(fa.at[pl.ds(_F_Q, _QPW)], [spl]),
                plsc.load_gather(fa.at[pl.ds(_F_Q + _QPW, _QPW)], [spl]),
                plsc.load_gather(fa.at[pl.ds(_F_Q + 2 * _QPW, _QPW)], [spl]),
                plsc.load_gather(ia.at[pl.ds(_I_QC, _QPW)], [spl]),
                jnp.full((16,), sbase + 31, jnp.int32)))
            ccs0.append(jnp.full((16,), sbase - 1, jnp.int32))
            ia[pl.ds(sbase, 16)] = zeros16

        def chunk(j, ccs):
            out = []
            for u in range(unroll):
                off = j * (16 * unroll) + u * 16
                px = fa[pl.ds(_F_PTX + off, 16)]
                py = fa[pl.ds(_F_PTY + off, 16)]
                pz = fa[pl.ds(_F_PTZ + off, 16)]
                pc = ia[pl.ds(_I_PTC + off, 16)]
                gidx = lane + off
                out = []
                for (qx, qy, qz, qc, cap_v), ccb in zip(qxyzc, ccs):
                    dx = px - qx
                    dy = py - qy
                    dz = pz - qz
                    d2 = dx * dx + dy * dy + dz * dz
                    thr = jnp.where(pc == qc, _R2, _R2P).astype(jnp.float32)
                    elig = d2 < thr
                    incl = plsc.cumsum(ones16, mask=elig)
                    plsc.store_scatter(ia, [ccb + incl], gidx, mask=elig)
                    ccb = jnp.minimum(
                        ccb + plsc.all_reduce_population_count(elig), cap_v)
                    out.append(ccb)
                ccs = tuple(out)
            return ccs

        ccs = lax.fori_loop(0, _N // (16 * unroll), chunk, tuple(ccs0))
        for q, (qx, qy, qz, qc, _cap), ccb in zip(qs, qxyzc, ccs):
            base = q * _S
            sbase = _I_WIDE + q * 64
            cntc = ccb - jnp.full((16,), sbase - 1, jnp.int32)
            pad = plsc.load_gather(ia, [jnp.full((16,), sbase, jnp.int32)])
            for h in range(2):
                lids = lane + h * 16
                iv = jnp.where(lids >= cntc,
                               pad, ia[pl.ds(sbase + h * 16, 16)])
                ia[pl.ds(_I_IDXC + base + h * 16, 16)] = iv
                fa[pl.ds(_F_STD + base + h * 16, 16)] = (
                    plsc.load_gather(fa.at[pl.ds(_F_PTX, _N)], [iv]) - qx)
                fa[pl.ds(_F_STD + _SPAN + base + h * 16, 16)] = (
                    plsc.load_gather(fa.at[pl.ds(_F_PTY, _N)], [iv]) - qy)
                fa[pl.ds(_F_STD + 2 * _SPAN + base + h * 16, 16)] = (
                    plsc.load_gather(fa.at[pl.ds(_F_PTZ, _N)], [iv]) - qz)
        return carry

    lax.fori_loop(0, stride, per_query, 0)

    w0 = lb * _PS + grp * _SPAN
    pltpu.sync_copy(ia.at[pl.ds(_I_IDXC, _SPAN)], sh_idx.at[pl.ds(w0, _SPAN)])
    for d in range(3):
        pltpu.sync_copy(
            fa.at[pl.ds(_F_STD + d * _SPAN, _SPAN)],
            out_hbm.at[pl.ds((b * _NCH + d) * _PS + grp * _SPAN, _SPAN)])
    plsc.subcore_barrier()

    c0 = grp * _WPC
    nchunks = _PS // _CH
    osems = (osem0, osem1)
    isems = (isem0, isem1)
    chbase = [jnp.full((16,), ch * _N, jnp.int32) for ch in range(4)]
    pending_out = [None, None]
    pending_idx = [None, None]

    def idx_copy(k):
        return pltpu.make_async_copy(
            sh_idx.at[pl.ds(lb * _PS + k * _CH, _CH)],
            ia.at[pl.ds(_I_IB[k % 2], _CH)], isems[k % 2])

    for chalf in range(2):
        ch0 = c0 + chalf * 4
        pltpu.sync_copy(feat_hbm.at[pl.ds((b * _C + ch0) * _N, 4 * _N)],
                        fa.at[pl.ds(_F_TABS, 4 * _N)])
        if chalf == 0:
            pending_idx[0] = idx_copy(0)
            pending_idx[0].start()
        for k in range(nchunks):
            buf = k % 2
            k0 = k * _CH
            ib = _I_IB[buf]
            stg = _F_STG + buf * 4 * _CH
            nk = k + 1 if k + 1 < nchunks else (0 if chalf == 0 else None)
            if nk is not None:
                pending_idx[nk % 2] = idx_copy(nk)
                pending_idx[nk % 2].start()
            pending_idx[buf].wait()
            if pending_out[buf] is not None:
                for c in pending_out[buf]:
                    c.wait()

            def inner(t, c2, ib=ib, stg=stg):
                for u in range(4):
                    o = t * 64 + u * 16
                    iv = ia[pl.ds(ib + o, 16)]
                    for ch in range(4):
                        fa[pl.ds(stg + ch * _CH + o, 16)] = (
                            plsc.load_gather(fa.at[pl.ds(_F_TABS, 4 * _N)],
                                             [chbase[ch] + iv]))
                return c2

            lax.fori_loop(0, _CH // 64, inner, 0)
            outs = []
            for ch in range(4):
                c = pltpu.make_async_copy(
                    fa.at[pl.ds(stg + ch * _CH, _CH)],
                    out_hbm.at[pl.ds(
                        (b * _NCH + 3 + ch0 + ch) * _PS + k0, _CH)],
                    osems[buf])
                c.start()
                outs.append(c)
            pending_out[buf] = outs
        for buf2 in range(2):
            if pending_out[buf2] is not None:
                for c in pending_out[buf2]:
                    c.wait()
                pending_out[buf2] = None


def kernel(xyz, new_xyz, components, new_components, features):
    xyz32 = xyz.astype(jnp.float32)
    nxyz32 = new_xyz.astype(jnp.float32)
    xs = xyz32[:, :, 0].reshape(-1)
    ys = xyz32[:, :, 1].reshape(-1)
    zs = xyz32[:, :, 2].reshape(-1)
    nxs = nxyz32[:, :, 0].reshape(-1)
    nys = nxyz32[:, :, 1].reshape(-1)
    nzs = nxyz32[:, :, 2].reshape(-1)
    comp = components.reshape(-1).astype(jnp.int32)
    ncomp = new_components.reshape(-1).astype(jnp.int32)
    out = _gag(xs, ys, zs, comp, nxs, nys, nzs, ncomp,
               features.astype(jnp.float32).reshape(-1))
    return out.reshape(_B, _NCH, _P, _S)


# speedup vs baseline: 1.0308x; 1.0098x over previous
import functools

import jax
import jax.numpy as jnp
from jax import lax
from jax.experimental import pallas as pl
from jax.experimental.pallas import tpu as pltpu
from jax.experimental.pallas import tpu_sc as plsc

_B, _N, _P, _C, _S = 4, 8192, 2048, 64, 32
_R2 = 0.2 * 0.2
_R2P = (0.5 * 0.2) ** 2
_WPB = 8
_QPW = _P // _WPB
_WPC = _C // _WPB
_PS = _P * _S
_NCH = 3 + _C
_CH = 4096
_SPAN = _QPW * _S

_F_PTX = 0
_F_PTY = _N
_F_PTZ = 2 * _N
_F_STD = 3 * _N
_F_Q = 3 * _N + 3 * _SPAN
_F_TABS = 0
_F_STG = 4 * _N
_F_SIZE = max(3 * _N + 3 * _SPAN + 3 * _QPW, 4 * _N + 2 * 4 * _CH)

_I_PTC = 0
_I_QC = _N
_I_WIDE = _N + _QPW
_I_IDXC = _N + _QPW + 64 * _QPW
_I_SIZE = _N + _QPW + 64 * _QPW + _SPAN
_I_IB = (0, _CH)

_mesh = plsc.VectorSubcoreMesh(core_axis_name="c", subcore_axis_name="s")


@functools.partial(
    pl.kernel,
    out_type=jax.ShapeDtypeStruct((_B * _NCH * _PS,), jnp.float32),
    mesh=_mesh,
    compiler_params=pltpu.CompilerParams(needs_layout_passes=False),
    scratch_types=[
        pltpu.VMEM((_F_SIZE,), jnp.float32),
        pltpu.VMEM((_I_SIZE,), jnp.int32),
        pltpu.VMEM_SHARED((2 * _PS,), jnp.int32),
        pltpu.SemaphoreType.DMA,
        pltpu.SemaphoreType.DMA,
        pltpu.SemaphoreType.DMA,
        pltpu.SemaphoreType.DMA,
    ],
)
def _gag(xs_hbm, ys_hbm, zs_hbm, comp_hbm, nxs_hbm, nys_hbm, nzs_hbm,
         ncomp_hbm, feat_hbm, out_hbm, fa, ia, sh_idx,
         osem0, osem1, isem0, isem1):
    cid = lax.axis_index("c")
    sid = lax.axis_index("s")
    lb = sid // _WPB
    b = cid * 2 + lb
    grp = sid % _WPB
    q0 = b * _P + grp * _QPW

    in_copies = [
        pltpu.make_async_copy(xs_hbm.at[pl.ds(b * _N, _N)],
                              fa.at[pl.ds(_F_PTX, _N)], isem0),
        pltpu.make_async_copy(ys_hbm.at[pl.ds(b * _N, _N)],
                              fa.at[pl.ds(_F_PTY, _N)], isem0),
        pltpu.make_async_copy(zs_hbm.at[pl.ds(b * _N, _N)],
                              fa.at[pl.ds(_F_PTZ, _N)], isem0),
        pltpu.make_async_copy(comp_hbm.at[pl.ds(b * _N, _N)],
                              ia.at[pl.ds(_I_PTC, _N)], isem0),
        pltpu.make_async_copy(nxs_hbm.at[pl.ds(q0, _QPW)],
                              fa.at[pl.ds(_F_Q, _QPW)], isem0),
        pltpu.make_async_copy(nys_hbm.at[pl.ds(q0, _QPW)],
                              fa.at[pl.ds(_F_Q + _QPW, _QPW)], isem0),
        pltpu.make_async_copy(nzs_hbm.at[pl.ds(q0, _QPW)],
                              fa.at[pl.ds(_F_Q + 2 * _QPW, _QPW)], isem0),
        pltpu.make_async_copy(ncomp_hbm.at[pl.ds(q0, _QPW)],
                              ia.at[pl.ds(_I_QC, _QPW)], isem0),
    ]
    for c in in_copies:
        c.start()
    for c in in_copies:
        c.wait()

    lane = lax.iota(jnp.int32, 16)
    zeros16 = jnp.zeros((16,), jnp.int32)
    ones16 = jnp.ones((16,), jnp.int32)
    unroll = 8
    qpack = 8
    stride = _QPW // qpack

    def per_query(p, carry):
        qs = tuple(p + i * stride for i in range(qpack))
        qxyzc = []
        ccs0 = []
        for q in qs:
            spl = jnp.full((16,), q, jnp.int32)
            sbase = _I_WIDE + q * 64
            qxyzc.append((
                plsc.load_gather(fa.at[pl.ds(_F_Q, _QPW)], [spl]),
                plsc.load_gather(fa.at[pl.ds(_F_Q + _QPW, _QPW)], [spl]),
                plsc.load_gather(fa.at[pl.ds(_F_Q + 2 * _QPW, _QPW)], [spl]),
                plsc.load_gather(ia.at[pl.ds(_I_QC, _QPW)], [spl]),
                jnp.full((16,), sbase + 31, jnp.int32)))
            ccs0.append(jnp.full((16,), sbase - 1, jnp.int32))
            ia[pl.ds(sbase, 16)] = zeros16

        def chunk(j, ccs):
            out = []
            for u in range(unroll):
                off = j * (16 * unroll) + u * 16
                px = fa[pl.ds(_F_PTX + off, 16)]
                py = fa[pl.ds(_F_PTY + off, 16)]
                pz = fa[pl.ds(_F_PTZ + off, 16)]
                pc = ia[pl.ds(_I_PTC + off, 16)]
                gidx = lane + off
                out = []
                for (qx, qy, qz, qc, cap_v), ccb in zip(qxyzc, ccs):
                    dx = px - qx
                    dy = py - qy
                    dz = pz - qz
                    d2 = dx * dx + dy * dy + dz * dz
                    thr = jnp.where(pc == qc, _R2, _R2P).astype(jnp.float32)
                    elig = d2 < thr
                    incl = plsc.cumsum(ones16, mask=elig)
                    plsc.store_scatter(ia, [ccb + incl], gidx, mask=elig)
                    ccb = jnp.minimum(
                        ccb + plsc.all_reduce_population_count(elig), cap_v)
                    out.append(ccb)
                ccs = tuple(out)
            return ccs

        ccs = lax.fori_loop(0, _N // (16 * unroll), chunk, tuple(ccs0))
        for q, (qx, qy, qz, qc, _cap), ccb in zip(qs, qxyzc, ccs):
            base = q * _S
            sbase = _I_WIDE + q * 64
            cntc = ccb - jnp.full((16,), sbase - 1, jnp.int32)
            pad = plsc.load_gather(ia, [jnp.full((16,), sbase, jnp.int32)])
            for h in range(2):
                lids = lane + h * 16
                iv = jnp.where(lids >= cntc,
                               pad, ia[pl.ds(sbase + h * 16, 16)])
                ia[pl.ds(_I_IDXC + base + h * 16, 16)] = iv
                fa[pl.ds(_F_STD + base + h * 16, 16)] = (
                    plsc.load_gather(fa.at[pl.ds(_F_PTX, _N)], [iv]) - qx)
                fa[pl.ds(_F_STD + _SPAN + base + h * 16, 16)] = (
                    plsc.load_gather(fa.at[pl.ds(_F_PTY, _N)], [iv]) - qy)
                fa[pl.ds(_F_STD + 2 * _SPAN + base + h * 16, 16)] = (
                    plsc.load_gather(fa.at[pl.ds(_F_PTZ, _N)], [iv]) - qz)
        return carry

    lax.fori_loop(0, stride, per_query, 0)

    w0 = lb * _PS + grp * _SPAN
    pltpu.sync_copy(ia.at[pl.ds(_I_IDXC, _SPAN)], sh_idx.at[pl.ds(w0, _SPAN)])
    for d in range(3):
        pltpu.sync_copy(
            fa.at[pl.ds(_F_STD + d * _SPAN, _SPAN)],
            out_hbm.at[pl.ds((b * _NCH + d) * _PS + grp * _SPAN, _SPAN)])
    plsc.subcore_barrier()

    c0 = grp * _WPC
    nchunks = _PS // _CH
    osems = (osem0, osem1)
    isems = (isem0, isem1)
    chbase = [jnp.full((16,), ch * _N, jnp.int32) for ch in range(4)]
    pending_out = [None, None]
    pending_idx = [None, None]

    def idx_copy(k):
        return pltpu.make_async_copy(
            sh_idx.at[pl.ds(lb * _PS + k * _CH, _CH)],
            ia.at[pl.ds(_I_IB[k % 2], _CH)], isems[k % 2])

    for chalf in range(2):
        ch0 = c0 + chalf * 4
        pltpu.sync_copy(feat_hbm.at[pl.ds((b * _C + ch0) * _N, 4 * _N)],
                        fa.at[pl.ds(_F_TABS, 4 * _N)])
        if chalf == 0:
            pending_idx[0] = idx_copy(0)
            pending_idx[0].start()
        for k in range(nchunks):
            buf = k % 2
            k0 = k * _CH
            ib = _I_IB[buf]
            stg = _F_STG + buf * 4 * _CH
            nk = k + 1 if k + 1 < nchunks else (0 if chalf == 0 else None)
            if nk is not None:
                pending_idx[nk % 2] = idx_copy(nk)
                pending_idx[nk % 2].start()
            pending_idx[buf].wait()
            if pending_out[buf] is not None:
                for c in pending_out[buf]:
                    c.wait()

            def inner(t, c2, ib=ib, stg=stg):
                for u in range(4):
                    o = t * 64 + u * 16
                    iv = ia[pl.ds(ib + o, 16)]
                    for ch in range(4):
                        fa[pl.ds(stg + ch * _CH + o, 16)] = (
                            plsc.load_gather(fa.at[pl.ds(_F_TABS, 4 * _N)],
                                             [chbase[ch] + iv]))
                return c2

            lax.fori_loop(0, _CH // 64, inner, 0)
            outs = []
            for ch in range(4):
                c = pltpu.make_async_copy(
                    fa.at[pl.ds(stg + ch * _CH, _CH)],
                    out_hbm.at[pl.ds(
                        (b * _NCH + 3 + ch0 + ch) * _PS + k0, _CH)],
                    osems[buf])
                c.start()
                outs.append(c)
            pending_out[buf] = outs
        for buf2 in range(2):
            if pending_out[buf2] is not None:
                for c in pending_out[buf2]:
                    c.wait()
                pending_out[buf2] = None


def kernel(xyz, new_xyz, components, new_components, features):
    xyz32 = xyz.astype(jnp.float32)
    nxyz32 = new_xyz.astype(jnp.float32)
    xs = xyz32[:, :, 0].reshape(-1)
    ys = xyz32[:, :, 1].reshape(-1)
    zs = xyz32[:, :, 2].reshape(-1)
    nxs = nxyz32[:, :, 0].reshape(-1)
    nys = nxyz32[:, :, 1].reshape(-1)
    nzs = nxyz32[:, :, 2].reshape(-1)
    comp = components.reshape(-1).astype(jnp.int32)
    ncomp = new_components.reshape(-1).astype(jnp.int32)
    out = _gag(xs, ys, zs, comp, nxs, nys, nzs, ncomp,
               features.astype(jnp.float32).reshape(-1))
    return out.reshape(_B, _NCH, _P, _S)
